# baseline (speedup 1.0000x reference)
import math

import jax
import jax.numpy as jnp
from jax.experimental import pallas as pl
from jax.experimental.pallas import tpu as pltpu

_BLK = 512
_FINE = 64
_SLOTS = 4


def _compute_block(fine_ref, dst_ref, p0):
    h = fine_ref.shape[1] // 2
    sub = _BLK // _FINE
    col = jax.lax.broadcasted_iota(jnp.int32, (sub, h), 1).astype(jnp.float32)
    row = jax.lax.broadcasted_iota(jnp.int32, (sub, h), 0).astype(jnp.float32)
    freq = jnp.exp((col * (1.0 / h)) * (-math.log(10000.0)))
    ang = (jnp.float32(p0) + row * jnp.float32(_FINE)) * freq
    cs = jnp.sin(ang)[:, None, :]
    cc = jnp.cos(ang)[:, None, :]
    fs = fine_ref[:, :h][None, :, :]
    fc = fine_ref[:, h:][None, :, :]
    dst_ref[:, :] = jnp.concatenate(
        [
            (fs * cc + fc * cs).reshape(_BLK, h),
            (fc * cc - fs * cs).reshape(_BLK, h),
        ],
        axis=1,
    )


def _make_body(seq_len):
    nblk = seq_len // _BLK

    def body(fine_ref, out_ref, scratch_ref, sem_ref):
        def copy(i):
            return pltpu.make_async_copy(
                scratch_ref.at[i % _SLOTS],
                out_ref.at[0, pl.ds(i * _BLK, _BLK), :],
                sem_ref.at[i % _SLOTS],
            )

        for i in range(nblk):
            if i >= _SLOTS:
                copy(i - _SLOTS).wait()
            _compute_block(fine_ref, scratch_ref.at[i % _SLOTS], i * _BLK)
            copy(i).start()
        for i in range(max(0, nblk - _SLOTS), nblk):
            copy(i).wait()

    return body


def kernel(x, emb):
    seq_len = x.shape[1]
    hidden = emb.shape[1]
    return pl.pallas_call(
        _make_body(seq_len),
        grid=(1,),
        in_specs=[
            pl.BlockSpec((_FINE, hidden), lambda i: (0, 0)),
        ],
        out_specs=pl.BlockSpec(memory_space=pl.ANY),
        out_shape=jax.ShapeDtypeStruct((1, seq_len, hidden), emb.dtype),
        scratch_shapes=[
            pltpu.VMEM((_SLOTS, _BLK, hidden), jnp.float32),
            pltpu.SemaphoreType.DMA((_SLOTS,)),
        ],
    )(emb)
